# Initial kernel scaffold; baseline (speedup 1.0000x reference)
import jax
import jax.numpy as jnp
from jax.experimental import pallas as pl


def kernel(x, edge_index, W1, b1, W2, b2):
    raise NotImplementedError("write your pallas kernel here")


# speedup vs baseline: 9.5272x; 9.5272x over previous
import functools

import jax
import jax.numpy as jnp
from jax import lax
from jax.experimental import pallas as pl
from jax.experimental.pallas import tpu as pltpu
from jax.experimental.pallas import tpu_sc as plsc

N = 10000
D = 128
E = 320000

NC = 2
NS = 16
NW = NC * NS
LANES = 16

NPAD = 10240
DUMMY = N
CH = 128
NCH = 79
EPT = NCH * CH
EPAD = NW * EPT
ZROWS = NPAD // NS

_mesh = plsc.VectorSubcoreMesh(
    core_axis_name="c", subcore_axis_name="s", num_cores=NC, num_subcores=NS)
_sc_params = pltpu.CompilerParams(needs_layout_passes=False)


@functools.partial(
    pl.kernel,
    out_type=jax.ShapeDtypeStruct((NW, NPAD), jnp.float32),
    mesh=_mesh,
    scratch_types=[
        pltpu.VMEM((EPT,), jnp.int32),
        pltpu.VMEM((NPAD,), jnp.float32),
    ],
    compiler_params=_sc_params,
)
def _deg_kernel(dst_hbm, out_hbm, dst_v, hist_v):
    c = lax.axis_index("c")
    s = lax.axis_index("s")
    wid = s * NC + c
    pltpu.sync_copy(dst_hbm.at[wid], dst_v)
    zeros16 = jnp.zeros((LANES,), jnp.float32)
    ones16 = jnp.ones((LANES,), jnp.float32)

    def zbody(i, carry):
        hist_v[pl.ds(i * LANES, LANES)] = zeros16
        return carry

    lax.fori_loop(0, NPAD // LANES, zbody, 0)

    def ebody(i, carry):
        idx = dst_v[pl.ds(i * LANES, LANES)]
        plsc.addupdate_scatter(hist_v, [idx], ones16)
        return carry

    lax.fori_loop(0, EPT // LANES, ebody, 0)
    pltpu.sync_copy(hist_v, out_hbm.at[wid])


@functools.partial(
    pl.kernel,
    out_type=jax.ShapeDtypeStruct((NC, NPAD, D), jnp.float32),
    mesh=_mesh,
    scratch_types=[
        pltpu.VMEM((NCH, CH), jnp.int32),
        pltpu.VMEM((NCH, CH), jnp.int32),
        pltpu.VMEM((CH, D), jnp.float32),
        pltpu.VMEM_SHARED((NPAD, D), jnp.float32),
        pltpu.SemaphoreType.DMA,
    ],
    compiler_params=_sc_params,
)
def _prop_kernel(table_hbm, src_hbm, dst_hbm, out_hbm,
                 src_v, dst_v, rows_v, accum, sem):
    c = lax.axis_index("c")
    s = lax.axis_index("s")
    wid = s * NC + c
    pltpu.sync_copy(src_hbm.at[wid], src_v)
    pltpu.sync_copy(dst_hbm.at[wid], dst_v)

    zeros16 = jnp.zeros((LANES,), jnp.float32)

    def zbody(i, carry):
        rows_v[i // (D // LANES), pl.ds((i % (D // LANES)) * LANES, LANES)] = zeros16
        return carry

    lax.fori_loop(0, CH * (D // LANES), zbody, 0)
    for z in range(ZROWS // CH):
        pltpu.sync_copy(rows_v, accum.at[pl.ds(s * ZROWS + z * CH, CH)])
    plsc.subcore_barrier()

    def ebody(j, carry):
        pltpu.async_copy(table_hbm.at[src_v.at[j]], rows_v, sem).wait()
        pltpu.sync_copy(rows_v, accum.at[dst_v.at[j]], add=True)
        return carry

    lax.fori_loop(0, NCH, ebody, 0)
    plsc.subcore_barrier()
    pltpu.sync_copy(accum.at[pl.ds(s * ZROWS, ZROWS)],
                    out_hbm.at[c].at[pl.ds(s * ZROWS, ZROWS)])


def _diag_scale(dis2, mat):
    ri = lax.broadcasted_iota(jnp.int32, (D, D), 0)
    ci = lax.broadcasted_iota(jnp.int32, (D, D), 1)
    diag = jnp.where(ri == ci, jnp.broadcast_to(dis2, (D, D)), 0.0)
    return jnp.dot(diag, mat, preferred_element_type=jnp.float32)


def _scale1_body(x_ref, pd_ref, w_ref, h_ref, dis_ref):
    pd = pd_ref[...][0]
    deg = jnp.sum(pd, axis=0, keepdims=True)
    dis = jnp.where(deg > 0.0, lax.rsqrt(jnp.maximum(deg, 1.0)), 0.0)
    xs = _diag_scale(dis, x_ref[...])
    h_ref[...] = jnp.dot(xs, w_ref[...], preferred_element_type=jnp.float32)
    dis_ref[...] = dis.reshape(1, 1, D)


def _scale2_body(p_ref, dis_ref, w_ref, b_ref, h_ref):
    p = p_ref[...]
    psum = p[0] + p[1]
    dis = dis_ref[...].reshape(1, D)
    h1 = _diag_scale(dis, psum) + b_ref[...]
    h_ref[...] = jnp.dot(_diag_scale(dis, h1), w_ref[...],
                         preferred_element_type=jnp.float32)


def _final_body(p_ref, dis_ref, b_ref, out_ref):
    p = p_ref[...]
    psum = p[0] + p[1]
    dis = dis_ref[...].reshape(1, D)
    out_ref[...] = _diag_scale(dis, psum) + b_ref[...]


BLK = 128
GRID = NPAD // BLK


def kernel(x, edge_index, W1, b1, W2, b2):
    src = edge_index[0]
    dst = edge_index[1]
    pad = EPAD - E
    src_p = jnp.concatenate([src, jnp.zeros((pad,), jnp.int32)])
    dst_p = jnp.concatenate([dst, jnp.full((pad,), DUMMY, jnp.int32)])
    src3 = src_p.reshape(NW, NCH, CH)
    dst3 = dst_p.reshape(NW, NCH, CH)
    dst2 = dst_p.reshape(NW, EPT)
    x_pad = jnp.pad(x, ((0, NPAD - N), (0, 0)))
    b1r = b1.reshape(1, D)
    b2r = b2.reshape(1, D)

    pd = _deg_kernel(dst2)
    pd3 = pd.reshape(NW, GRID, BLK).transpose(1, 0, 2)

    h1s, dis3 = pl.pallas_call(
        _scale1_body,
        grid=(GRID,),
        in_specs=[
            pl.BlockSpec((BLK, D), lambda i: (i, 0)),
            pl.BlockSpec((1, NW, BLK), lambda i: (i, 0, 0)),
            pl.BlockSpec((D, D), lambda i: (0, 0)),
        ],
        out_specs=[
            pl.BlockSpec((BLK, D), lambda i: (i, 0)),
            pl.BlockSpec((1, 1, BLK), lambda i: (i, 0, 0)),
        ],
        out_shape=[
            jax.ShapeDtypeStruct((NPAD, D), jnp.float32),
            jax.ShapeDtypeStruct((GRID, 1, BLK), jnp.float32),
        ],
    )(x_pad, pd3, W1)

    p1 = _prop_kernel(h1s, src3, dst3)

    h2s = pl.pallas_call(
        _scale2_body,
        grid=(GRID,),
        in_specs=[
            pl.BlockSpec((NC, BLK, D), lambda i: (0, i, 0)),
            pl.BlockSpec((1, 1, BLK), lambda i: (i, 0, 0)),
            pl.BlockSpec((D, D), lambda i: (0, 0)),
            pl.BlockSpec((1, D), lambda i: (0, 0)),
        ],
        out_specs=pl.BlockSpec((BLK, D), lambda i: (i, 0)),
        out_shape=jax.ShapeDtypeStruct((NPAD, D), jnp.float32),
    )(p1, dis3, W2, b1r)

    p2 = _prop_kernel(h2s, src3, dst3)

    out = pl.pallas_call(
        _final_body,
        grid=(GRID,),
        in_specs=[
            pl.BlockSpec((NC, BLK, D), lambda i: (0, i, 0)),
            pl.BlockSpec((1, 1, BLK), lambda i: (i, 0, 0)),
            pl.BlockSpec((1, D), lambda i: (0, 0)),
        ],
        out_specs=pl.BlockSpec((BLK, D), lambda i: (i, 0)),
        out_shape=jax.ShapeDtypeStruct((NPAD, D), jnp.float32),
    )(p2, dis3, b2r)

    return out[:N]
